# baseline (device time: 16274 ns/iter reference)
import jax
import jax.numpy as jnp
from jax import lax
from jax.experimental import pallas as pl
from jax.experimental.pallas import tpu as pltpu

HALF = 256
CH = 32
NCH = HALF // CH
BLK = 64
NBLK = HALF // BLK
CPB = BLK // CH


def kernel(dy, W):
    m, _ = dy.shape
    d = W.shape[0]

    def body(dy_ref, w_ref, out_ref, wbuf, sbuf, rbuf, zbuf, pbuf,
             zs_sems, zr_sems, ps_sems, pr_sems):
        my_x = lax.axis_index("x")
        my_y = lax.axis_index("y")
        my_z = lax.axis_index("z")
        z_nbr = (my_x, my_y, 1 - my_z)
        x_nbr = (1 - my_x, my_y, my_z)
        y_nbr = (my_x, 1 - my_y, my_z)

        m_lo = jnp.where(my_x == my_y, 0, HALF)
        o_lo = HALF - m_lo

        barrier = pltpu.get_barrier_semaphore()
        for nbr in (z_nbr, x_nbr, y_nbr):
            pl.semaphore_signal(
                barrier, inc=1, device_id=nbr,
                device_id_type=pl.DeviceIdType.MESH,
            )

        z_rdma = [
            pltpu.make_async_remote_copy(
                src_ref=sbuf.at[pl.ds(c * CH, CH), :],
                dst_ref=zbuf.at[c],
                send_sem=zs_sems.at[c],
                recv_sem=zr_sems.at[c],
                device_id=z_nbr,
                device_id_type=pl.DeviceIdType.MESH,
            )
            for c in range(NCH)
        ]
        p_rdma = [
            pltpu.make_async_remote_copy(
                src_ref=rbuf.at[pl.ds(c * CH, CH), :],
                dst_ref=pbuf.at[c],
                send_sem=ps_sems.at[c],
                recv_sem=pr_sems.at[c],
                device_id=x_nbr if c < NCH // 2 else y_nbr,
                device_id_type=pl.DeviceIdType.MESH,
            )
            for c in range(NCH)
        ]

        wbuf[...] = w_ref[...].astype(jnp.bfloat16)

        for h in range(NBLK):
            rows = pl.ds(h * BLK, BLK)
            part = lax.dot_general(
                dy_ref[pl.ds(m_lo + h * BLK, BLK), :].astype(jnp.bfloat16),
                wbuf[...],
                dimension_numbers=(((1,), (1,)), ((), ())),
                preferred_element_type=jnp.float32,
            )
            out_ref[pl.ds(m_lo + h * BLK, BLK), :] = part
            sbuf[rows, :] = part.astype(jnp.bfloat16)
            if h == 0:
                pl.semaphore_wait(barrier, 3)
            for c in range(CPB * h, CPB * (h + 1)):
                z_rdma[c].start()

        for c in range(NCH):
            z_rdma[c].wait_recv()
            rows = pl.ds(m_lo + c * CH, CH)
            red = out_ref[rows, :] + zbuf[c, :, :].astype(jnp.float32)
            out_ref[rows, :] = red
            rbuf[pl.ds(c * CH, CH), :] = red.astype(jnp.bfloat16)
            p_rdma[c].start()

        for c in range(NCH):
            p_rdma[c].wait_recv()
            out_ref[pl.ds(o_lo + c * CH, CH), :] = (
                pbuf[c, :, :].astype(jnp.float32)
            )

        for c in range(NCH):
            z_rdma[c].wait_send()
            p_rdma[c].wait_send()

    return pl.pallas_call(
        body,
        out_shape=jax.ShapeDtypeStruct((m, d), jnp.float32),
        in_specs=[
            pl.BlockSpec(memory_space=pltpu.VMEM),
            pl.BlockSpec(memory_space=pltpu.VMEM),
        ],
        out_specs=pl.BlockSpec(memory_space=pltpu.VMEM),
        scratch_shapes=[
            pltpu.VMEM((d, 2048), jnp.bfloat16),
            pltpu.VMEM((HALF, d), jnp.bfloat16),
            pltpu.VMEM((HALF, d), jnp.bfloat16),
            pltpu.VMEM((NCH, CH, d), jnp.bfloat16),
            pltpu.VMEM((NCH, CH, d), jnp.bfloat16),
            pltpu.SemaphoreType.DMA((NCH,)),
            pltpu.SemaphoreType.DMA((NCH,)),
            pltpu.SemaphoreType.DMA((NCH,)),
            pltpu.SemaphoreType.DMA((NCH,)),
        ],
        compiler_params=pltpu.CompilerParams(collective_id=0),
    )(dy, W)


# device time: 16013 ns/iter; 1.0163x vs baseline; 1.0163x over previous
import jax
import jax.numpy as jnp
from jax import lax
from jax.experimental import pallas as pl
from jax.experimental.pallas import tpu as pltpu

HALF = 256
CH = 32
NCH = HALF // CH
BLK = 64
NBLK = HALF // BLK
CPB = BLK // CH


def kernel(dy, W):
    m, _ = dy.shape
    d = W.shape[0]

    def body(dy_ref, w_ref, out_ref, sbuf, rbuf, zbuf, pbuf,
             zs_sems, zr_sems, ps_sems, pr_sems):
        my_x = lax.axis_index("x")
        my_y = lax.axis_index("y")
        my_z = lax.axis_index("z")
        z_nbr = (my_x, my_y, 1 - my_z)
        x_nbr = (1 - my_x, my_y, my_z)
        y_nbr = (my_x, 1 - my_y, my_z)

        m_lo = jnp.where(my_x == my_y, 0, HALF)
        o_lo = HALF - m_lo

        barrier = pltpu.get_barrier_semaphore()
        for nbr in (z_nbr, x_nbr, y_nbr):
            pl.semaphore_signal(
                barrier, inc=1, device_id=nbr,
                device_id_type=pl.DeviceIdType.MESH,
            )

        z_rdma = [
            pltpu.make_async_remote_copy(
                src_ref=sbuf.at[pl.ds(c * CH, CH), :],
                dst_ref=zbuf.at[c],
                send_sem=zs_sems.at[c],
                recv_sem=zr_sems.at[c],
                device_id=z_nbr,
                device_id_type=pl.DeviceIdType.MESH,
            )
            for c in range(NCH)
        ]
        p_rdma = [
            pltpu.make_async_remote_copy(
                src_ref=rbuf.at[pl.ds(c * CH, CH), :],
                dst_ref=pbuf.at[c],
                send_sem=ps_sems.at[c],
                recv_sem=pr_sems.at[c],
                device_id=x_nbr if c % 2 == 0 else y_nbr,
                device_id_type=pl.DeviceIdType.MESH,
            )
            for c in range(NCH)
        ]

        for h in range(NBLK):
            rows = pl.ds(h * BLK, BLK)
            part = lax.dot_general(
                dy_ref[pl.ds(m_lo + h * BLK, BLK), :],
                w_ref[...],
                dimension_numbers=(((1,), (1,)), ((), ())),
                preferred_element_type=jnp.float32,
            )
            out_ref[pl.ds(m_lo + h * BLK, BLK), :] = part
            sbuf[rows, :] = part.astype(jnp.bfloat16)
            if h == 0:
                pl.semaphore_wait(barrier, 3)
            for c in range(CPB * h, CPB * (h + 1)):
                z_rdma[c].start()

        for c in range(NCH):
            z_rdma[c].wait_recv()
            rows = pl.ds(m_lo + c * CH, CH)
            red = out_ref[rows, :] + zbuf[c, :, :].astype(jnp.float32)
            out_ref[rows, :] = red
            rbuf[pl.ds(c * CH, CH), :] = red.astype(jnp.bfloat16)
            p_rdma[c].start()

        for c in range(NCH):
            p_rdma[c].wait_recv()
            out_ref[pl.ds(o_lo + c * CH, CH), :] = (
                pbuf[c, :, :].astype(jnp.float32)
            )

        for c in range(NCH):
            z_rdma[c].wait_send()
            p_rdma[c].wait_send()

    return pl.pallas_call(
        body,
        out_shape=jax.ShapeDtypeStruct((m, d), jnp.float32),
        in_specs=[
            pl.BlockSpec(memory_space=pltpu.VMEM),
            pl.BlockSpec(memory_space=pltpu.VMEM),
        ],
        out_specs=pl.BlockSpec(memory_space=pltpu.VMEM),
        scratch_shapes=[
            pltpu.VMEM((HALF, d), jnp.bfloat16),
            pltpu.VMEM((HALF, d), jnp.bfloat16),
            pltpu.VMEM((NCH, CH, d), jnp.bfloat16),
            pltpu.VMEM((NCH, CH, d), jnp.bfloat16),
            pltpu.SemaphoreType.DMA((NCH,)),
            pltpu.SemaphoreType.DMA((NCH,)),
            pltpu.SemaphoreType.DMA((NCH,)),
            pltpu.SemaphoreType.DMA((NCH,)),
        ],
        compiler_params=pltpu.CompilerParams(collective_id=0),
    )(dy, W)


# device time: 14830 ns/iter; 1.0974x vs baseline; 1.0798x over previous
import jax
import jax.numpy as jnp
from jax import lax
from jax.experimental import pallas as pl
from jax.experimental.pallas import tpu as pltpu

HALF = 256
CH = 32
NCH = HALF // CH
BLK = 64
NBLK = HALF // BLK
CPB = BLK // CH


def kernel(dy, W):
    m, _ = dy.shape
    d = W.shape[0]

    mx = lax.axis_index("x")
    my = lax.axis_index("y")
    m_lo_val = jnp.where(mx == my, 0, HALF)
    dy_half = lax.dynamic_slice_in_dim(dy, m_lo_val, HALF, 0).astype(jnp.bfloat16)
    W_bf = W.astype(jnp.bfloat16)

    def body(dy_ref, w_ref, out_ref, sbuf, rbuf, zbuf, pbuf,
             zs_sems, zr_sems, ps_sems, pr_sems):
        my_x = lax.axis_index("x")
        my_y = lax.axis_index("y")
        my_z = lax.axis_index("z")
        z_nbr = (my_x, my_y, 1 - my_z)
        x_nbr = (1 - my_x, my_y, my_z)
        y_nbr = (my_x, 1 - my_y, my_z)

        m_lo = jnp.where(my_x == my_y, 0, HALF)
        o_lo = HALF - m_lo

        barrier = pltpu.get_barrier_semaphore()
        for nbr in (z_nbr, x_nbr, y_nbr):
            pl.semaphore_signal(
                barrier, inc=1, device_id=nbr,
                device_id_type=pl.DeviceIdType.MESH,
            )

        z_rdma = [
            pltpu.make_async_remote_copy(
                src_ref=sbuf.at[pl.ds(c * CH, CH), :],
                dst_ref=zbuf.at[c],
                send_sem=zs_sems.at[c],
                recv_sem=zr_sems.at[c],
                device_id=z_nbr,
                device_id_type=pl.DeviceIdType.MESH,
            )
            for c in range(NCH)
        ]
        p_rdma = [
            pltpu.make_async_remote_copy(
                src_ref=rbuf.at[pl.ds(c * CH, CH), :],
                dst_ref=pbuf.at[c],
                send_sem=ps_sems.at[c],
                recv_sem=pr_sems.at[c],
                device_id=x_nbr if c % 2 == 0 else y_nbr,
                device_id_type=pl.DeviceIdType.MESH,
            )
            for c in range(NCH)
        ]

        for h in range(NBLK):
            rows = pl.ds(h * BLK, BLK)
            part = lax.dot_general(
                dy_ref[rows, :],
                w_ref[...],
                dimension_numbers=(((1,), (1,)), ((), ())),
                preferred_element_type=jnp.float32,
            )
            out_ref[pl.ds(m_lo + h * BLK, BLK), :] = part
            sbuf[rows, :] = part.astype(jnp.bfloat16)
            if h == 0:
                pl.semaphore_wait(barrier, 3)
            for c in range(CPB * h, CPB * (h + 1)):
                z_rdma[c].start()

        for c in range(NCH):
            z_rdma[c].wait_recv()
            rows = pl.ds(m_lo + c * CH, CH)
            red = out_ref[rows, :] + zbuf[c, :, :].astype(jnp.float32)
            out_ref[rows, :] = red
            rbuf[pl.ds(c * CH, CH), :] = red.astype(jnp.bfloat16)
            p_rdma[c].start()

        for c in range(NCH):
            p_rdma[c].wait_recv()
            out_ref[pl.ds(o_lo + c * CH, CH), :] = (
                pbuf[c, :, :].astype(jnp.float32)
            )

        for c in range(NCH):
            z_rdma[c].wait_send()
            p_rdma[c].wait_send()

    return pl.pallas_call(
        body,
        out_shape=jax.ShapeDtypeStruct((m, d), jnp.float32),
        in_specs=[
            pl.BlockSpec(memory_space=pltpu.VMEM),
            pl.BlockSpec(memory_space=pltpu.VMEM),
        ],
        out_specs=pl.BlockSpec(memory_space=pltpu.VMEM),
        scratch_shapes=[
            pltpu.VMEM((HALF, d), jnp.bfloat16),
            pltpu.VMEM((HALF, d), jnp.bfloat16),
            pltpu.VMEM((NCH, CH, d), jnp.bfloat16),
            pltpu.VMEM((NCH, CH, d), jnp.bfloat16),
            pltpu.SemaphoreType.DMA((NCH,)),
            pltpu.SemaphoreType.DMA((NCH,)),
            pltpu.SemaphoreType.DMA((NCH,)),
            pltpu.SemaphoreType.DMA((NCH,)),
        ],
        compiler_params=pltpu.CompilerParams(collective_id=0),
    )(dy_half, W_bf)


# device time: 14729 ns/iter; 1.1049x vs baseline; 1.0069x over previous
import jax
import jax.numpy as jnp
from jax import lax
from jax.experimental import pallas as pl
from jax.experimental.pallas import tpu as pltpu

HALF = 256
CH = 32
NCH = HALF // CH
BLK = 64
NBLK = HALF // BLK
CPB = BLK // CH


def kernel(dy, W):
    m, _ = dy.shape
    d = W.shape[0]

    mx = lax.axis_index("x")
    my = lax.axis_index("y")
    m_lo_val = jnp.where(mx == my, 0, HALF)
    dy_half = lax.dynamic_slice_in_dim(dy, m_lo_val, HALF, 0).astype(jnp.bfloat16)
    W_bf = W.astype(jnp.bfloat16)

    def body(dy_ref, w_ref, out_ref, abuf, sbuf, zbuf, pbuf,
             zs_sems, zr_sems, ps_sems, pr_sems):
        my_x = lax.axis_index("x")
        my_y = lax.axis_index("y")
        my_z = lax.axis_index("z")
        z_nbr = (my_x, my_y, 1 - my_z)
        x_nbr = (1 - my_x, my_y, my_z)
        y_nbr = (my_x, 1 - my_y, my_z)

        m_lo = jnp.where(my_x == my_y, 0, HALF)
        o_lo = HALF - m_lo

        barrier = pltpu.get_barrier_semaphore()
        for nbr in (z_nbr, x_nbr, y_nbr):
            pl.semaphore_signal(
                barrier, inc=1, device_id=nbr,
                device_id_type=pl.DeviceIdType.MESH,
            )

        z_rdma = [
            pltpu.make_async_remote_copy(
                src_ref=sbuf.at[pl.ds(c * CH, CH), :],
                dst_ref=zbuf.at[c],
                send_sem=zs_sems.at[c],
                recv_sem=zr_sems.at[c],
                device_id=z_nbr,
                device_id_type=pl.DeviceIdType.MESH,
            )
            for c in range(NCH)
        ]
        p_rdma = [
            pltpu.make_async_remote_copy(
                src_ref=out_ref.at[pl.ds(m_lo + c * CH, CH), :],
                dst_ref=pbuf.at[c],
                send_sem=ps_sems.at[c],
                recv_sem=pr_sems.at[c],
                device_id=x_nbr if c % 2 == 0 else y_nbr,
                device_id_type=pl.DeviceIdType.MESH,
            )
            for c in range(NCH)
        ]

        for h in range(NBLK):
            rows = pl.ds(h * BLK, BLK)
            part = lax.dot_general(
                dy_ref[rows, :],
                w_ref[...],
                dimension_numbers=(((1,), (1,)), ((), ())),
                preferred_element_type=jnp.float32,
            )
            abuf[rows, :] = part
            sbuf[rows, :] = part.astype(jnp.bfloat16)
            if h == 0:
                pl.semaphore_wait(barrier, 3)
            for c in range(CPB * h, CPB * (h + 1)):
                z_rdma[c].start()

        for c in range(NCH):
            z_rdma[c].wait_recv()
            red_bf = (
                abuf[pl.ds(c * CH, CH), :] + zbuf[c, :, :].astype(jnp.float32)
            ).astype(jnp.bfloat16)
            out_ref[pl.ds(m_lo + c * CH, CH), :] = red_bf
            p_rdma[c].start()

        for c in range(NCH):
            p_rdma[c].wait_recv()
            out_ref[pl.ds(o_lo + c * CH, CH), :] = pbuf[c, :, :]

        for c in range(NCH):
            z_rdma[c].wait_send()
            p_rdma[c].wait_send()

    return pl.pallas_call(
        body,
        out_shape=jax.ShapeDtypeStruct((m, d), jnp.bfloat16),
        in_specs=[
            pl.BlockSpec(memory_space=pltpu.VMEM),
            pl.BlockSpec(memory_space=pltpu.VMEM),
        ],
        out_specs=pl.BlockSpec(memory_space=pltpu.VMEM),
        scratch_shapes=[
            pltpu.VMEM((HALF, d), jnp.float32),
            pltpu.VMEM((HALF, d), jnp.bfloat16),
            pltpu.VMEM((NCH, CH, d), jnp.bfloat16),
            pltpu.VMEM((NCH, CH, d), jnp.bfloat16),
            pltpu.SemaphoreType.DMA((NCH,)),
            pltpu.SemaphoreType.DMA((NCH,)),
            pltpu.SemaphoreType.DMA((NCH,)),
            pltpu.SemaphoreType.DMA((NCH,)),
        ],
        compiler_params=pltpu.CompilerParams(collective_id=0),
    )(dy_half, W_bf)


# device time: 14242 ns/iter; 1.1427x vs baseline; 1.0342x over previous
import jax
import jax.numpy as jnp
from jax import lax
from jax.experimental import pallas as pl
from jax.experimental.pallas import tpu as pltpu

HALF = 256
CH = 16
NCH = HALF // CH
BLK = 128
NBLK = HALF // BLK
CPB = BLK // CH


def kernel(dy, W):
    m, _ = dy.shape
    d = W.shape[0]

    mx = lax.axis_index("x")
    my = lax.axis_index("y")
    m_lo_val = jnp.where(mx == my, 0, HALF)
    dy_half = lax.dynamic_slice_in_dim(dy, m_lo_val, HALF, 0).astype(jnp.bfloat16)
    W_bf = W.astype(jnp.bfloat16)

    def body(dy_ref, w_ref, out_ref, abuf, sbuf, zbuf,
             zs_sems, zr_sems, ps_sems, pr_sems):
        my_x = lax.axis_index("x")
        my_y = lax.axis_index("y")
        my_z = lax.axis_index("z")
        z_nbr = (my_x, my_y, 1 - my_z)
        x_nbr = (1 - my_x, my_y, my_z)
        y_nbr = (my_x, 1 - my_y, my_z)

        m_lo = jnp.where(my_x == my_y, 0, HALF)
        o_lo = HALF - m_lo

        barrier = pltpu.get_barrier_semaphore()
        for nbr in (z_nbr, x_nbr, y_nbr):
            pl.semaphore_signal(
                barrier, inc=1, device_id=nbr,
                device_id_type=pl.DeviceIdType.MESH,
            )

        z_rdma = [
            pltpu.make_async_remote_copy(
                src_ref=sbuf.at[pl.ds(c * CH, CH), :],
                dst_ref=zbuf.at[c],
                send_sem=zs_sems.at[c],
                recv_sem=zr_sems.at[c],
                device_id=z_nbr,
                device_id_type=pl.DeviceIdType.MESH,
            )
            for c in range(NCH)
        ]
        p_rdma = [
            pltpu.make_async_remote_copy(
                src_ref=out_ref.at[pl.ds(m_lo + c * CH, CH), :],
                dst_ref=out_ref.at[pl.ds(m_lo + c * CH, CH), :],
                send_sem=ps_sems.at[c],
                recv_sem=pr_sems.at[c],
                device_id=x_nbr if c % 2 == 0 else y_nbr,
                device_id_type=pl.DeviceIdType.MESH,
            )
            for c in range(NCH)
        ]

        for h in range(NBLK):
            rows = pl.ds(h * BLK, BLK)
            part = lax.dot_general(
                dy_ref[rows, :],
                w_ref[...],
                dimension_numbers=(((1,), (1,)), ((), ())),
                preferred_element_type=jnp.float32,
            )
            abuf[rows, :] = part
            sbuf[rows, :] = part.astype(jnp.bfloat16)
            if h == 0:
                pl.semaphore_wait(barrier, 3)
            for c in range(CPB * h, CPB * (h + 1)):
                z_rdma[c].start()

        for c in range(NCH):
            z_rdma[c].wait_recv()
            red_bf = (
                abuf[pl.ds(c * CH, CH), :] + zbuf[c, :, :].astype(jnp.float32)
            ).astype(jnp.bfloat16)
            out_ref[pl.ds(m_lo + c * CH, CH), :] = red_bf
            p_rdma[c].start()

        for c in range(NCH):
            p_rdma[c].wait_recv()

        for c in range(NCH):
            z_rdma[c].wait_send()
            p_rdma[c].wait_send()

    return pl.pallas_call(
        body,
        out_shape=jax.ShapeDtypeStruct((m, d), jnp.bfloat16),
        in_specs=[
            pl.BlockSpec(memory_space=pltpu.VMEM),
            pl.BlockSpec(memory_space=pltpu.VMEM),
        ],
        out_specs=pl.BlockSpec(memory_space=pltpu.VMEM),
        scratch_shapes=[
            pltpu.VMEM((HALF, d), jnp.float32),
            pltpu.VMEM((HALF, d), jnp.bfloat16),
            pltpu.VMEM((NCH, CH, d), jnp.bfloat16),
            pltpu.SemaphoreType.DMA((NCH,)),
            pltpu.SemaphoreType.DMA((NCH,)),
            pltpu.SemaphoreType.DMA((NCH,)),
            pltpu.SemaphoreType.DMA((NCH,)),
        ],
        compiler_params=pltpu.CompilerParams(collective_id=0),
    )(dy_half, W_bf)
